# baseline (speedup 1.0000x reference)
import dataclasses

import jax
import jax.numpy as jnp
from jax import lax
from jax.experimental import pallas as pl
from jax.experimental.pallas import tpu as pltpu
from jax.experimental.pallas import tpu_sc as plsc

VOCAB = 1000000
EMBED = 64
N_CTX = 50
N_NEG = 20
LANES = 16
N_SUB = 16
CTX_SLOTS = 4
OUT_SLOTS = 2
SLOTS = CTX_SLOTS + OUT_SLOTS
N_PAD = N_SUB * SLOTS
NEG_BASE = N_SUB * CTX_SLOTS
TGT_ROW = NEG_BASE + 24
TILE_L = 128
INV_CTX = 1.0 / N_CTX


def _softplus16(t):
    m = jnp.maximum(t, 0.0)
    u = jnp.exp(-jnp.abs(t))
    y = 1.0 + u
    x = u * (6.0 + u) / (6.0 + 4.0 * u)
    for _ in range(3):
        x = x + y * jnp.exp(-x) - 1.0
    return m + x


def _sc_body(ctx_hbm, neg_hbm, tgt_hbm, wt_in_hbm, wt_out_hbm, out_hbm,
             idx_v, blks_v, col_v, final_v, acc_v, scores_v, out_v,
             stage_shr, sem_a, sem_b):
    cid = lax.axis_index("c")
    sub = lax.axis_index("s")

    @pl.when(cid == 0)
    def _():
        io = lax.iota(jnp.int32, LANES)
        zero16f = jnp.zeros((LANES,), jnp.float32)

        idx_v[pl.ds(N_CTX - 2, LANES)] = jnp.zeros((LANES,), jnp.int32)
        idx_v[pl.ds(NEG_BASE + LANES, LANES)] = jnp.zeros((LANES,), jnp.int32)
        cp_ctx = pltpu.async_copy(ctx_hbm, idx_v.at[pl.ds(0, N_CTX)], sem_b)
        cp_neg = pltpu.async_copy(neg_hbm, idx_v.at[pl.ds(NEG_BASE, N_NEG)],
                                  sem_b)
        cp_tgt = pltpu.async_copy(tgt_hbm, idx_v.at[pl.ds(TGT_ROW, 1)], sem_b)

        def _fire(s, src):
            chunk = idx_v[pl.ds(pl.multiple_of(LANES * s, LANES), LANES)]
            r = jnp.sum(jnp.where(io == sub, chunk, 0))
            q128 = pl.multiple_of((r // TILE_L) * TILE_L, TILE_L)
            pltpu.async_copy(src.at[:, pl.ds(q128, TILE_L)],
                             blks_v.at[s], sem_a)

        live_pred = {3: sub < 2, 5: jnp.logical_or(sub < 4, sub == 8)}

        cp_ctx.wait()

        @pl.loop(0, CTX_SLOTS - 1)
        def _(s):
            _fire(s, wt_in_hbm)

        @pl.when(live_pred[3])
        def _():
            _fire(3, wt_in_hbm)

        cp_neg.wait()
        cp_tgt.wait()
        _fire(4, wt_out_hbm)

        @pl.when(live_pred[5])
        def _():
            _fire(5, wt_out_hbm)

        def _drain(s):
            pltpu.make_async_copy(wt_in_hbm.at[:, pl.ds(0, TILE_L)],
                                  blks_v.at[s], sem_a).wait()

        @pl.loop(0, CTX_SLOTS - 1)
        def _(s):
            _drain(s)

        @pl.when(live_pred[3])
        def _():
            _drain(3)
        _drain(4)

        @pl.when(live_pred[5])
        def _():
            _drain(5)

        def _stage(s):
            chunk = idx_v[pl.ds(pl.multiple_of(LANES * s, LANES), LANES)]
            r = jnp.sum(jnp.where(io == sub, chunk, 0))
            q = r - (r // TILE_L) * TILE_L
            colidx = jnp.full((LANES,), q, jnp.int32)
            for c in range(EMBED // LANES):
                vals = plsc.load_gather(
                    blks_v.at[s], [io + LANES * c, colidx])
                col_v[s, pl.ds(LANES * c, LANES)] = vals
            pltpu.async_copy(col_v.at[s], stage_shr.at[sub + LANES * s],
                             sem_b)

        def _stage_drain(s):
            pltpu.make_async_copy(col_v.at[s],
                                  stage_shr.at[sub + LANES * s],
                                  sem_b).wait()

        @pl.loop(0, CTX_SLOTS - 1)
        def _(s):
            _stage(s)

        @pl.when(live_pred[3])
        def _():
            _stage(3)
        _stage(4)

        @pl.when(live_pred[5])
        def _():
            _stage(5)

        @pl.loop(0, CTX_SLOTS - 1)
        def _(s):
            _stage_drain(s)

        @pl.when(live_pred[3])
        def _():
            _stage_drain(3)
        _stage_drain(4)

        @pl.when(live_pred[5])
        def _():
            _stage_drain(5)

        plsc.subcore_barrier()

        @pl.when(sub == 0)
        def _():
            pltpu.sync_copy(stage_shr, final_v)

            for c in range(EMBED // LANES):
                acc_v[pl.ds(LANES * c, LANES)] = zero16f
            scores_v[pl.ds(0, LANES)] = zero16f
            scores_v[pl.ds(LANES, LANES)] = zero16f

            @pl.loop(0, N_CTX)
            def _(i):
                for c in range(EMBED // LANES):
                    sl = pl.ds(LANES * c, LANES)
                    acc_v[sl] = acc_v[sl] + final_v[i, sl]

            @pl.loop(0, N_NEG)
            def _(j):
                p = final_v[j + NEG_BASE, pl.ds(0, LANES)] * acc_v[pl.ds(0, LANES)]
                for c in range(1, EMBED // LANES):
                    sl = pl.ds(LANES * c, LANES)
                    p = p + final_v[j + NEG_BASE, sl] * acc_v[sl]
                s = jnp.sum(p) * INV_CTX
                plsc.store_scatter(scores_v, [jnp.full((LANES,), j, jnp.int32)],
                                   jnp.full((LANES,), s, jnp.float32),
                                   mask=io == 0)

            p = final_v[TGT_ROW, pl.ds(0, LANES)] * acc_v[pl.ds(0, LANES)]
            for c in range(1, EMBED // LANES):
                sl = pl.ds(LANES * c, LANES)
                p = p + final_v[TGT_ROW, sl] * acc_v[sl]
            s_pos = jnp.sum(p) * INV_CTX

            neg100 = jnp.full((LANES,), -100.0, jnp.float32)
            t0 = scores_v[pl.ds(0, LANES)]
            t1 = jnp.where(io < (N_NEG - LANES), scores_v[pl.ds(LANES, LANES)],
                           jnp.where(io == (N_NEG - LANES), -s_pos, neg100))
            loss = jnp.sum(_softplus16(t0)) + jnp.sum(_softplus16(t1))
            out_v[...] = jnp.full((LANES,), loss, jnp.float32)
            pltpu.sync_copy(out_v.at[pl.ds(0, 1)], out_hbm)


@jax.jit
def _cbow_loss(ctx, neg, tgt, wt_in, wt_out):
    mesh = plsc.VectorSubcoreMesh(core_axis_name="c", subcore_axis_name="s",
                                  num_cores=1, num_subcores=N_SUB)
    cp = pltpu.CompilerParams()
    if "needs_layout_passes" in pltpu.CompilerParams.__dataclass_fields__:
        cp = dataclasses.replace(cp, needs_layout_passes=False)
    cp = dataclasses.replace(cp, disable_bounds_checks=True)
    run = pl.kernel(
        _sc_body,
        out_type=jax.ShapeDtypeStruct((1,), jnp.float32),
        mesh=mesh,
        scratch_types=[
            pltpu.VMEM((N_PAD,), jnp.int32),
            pltpu.VMEM((SLOTS, EMBED, TILE_L), jnp.float32),
            pltpu.VMEM((SLOTS, EMBED), jnp.float32),
            pltpu.VMEM((N_PAD, EMBED), jnp.float32),
            pltpu.VMEM((EMBED,), jnp.float32),
            pltpu.VMEM((2 * LANES,), jnp.float32),
            pltpu.VMEM((LANES,), jnp.float32),
            pltpu.VMEM_SHARED((N_PAD, EMBED), jnp.float32),
            pltpu.SemaphoreType.DMA,
            pltpu.SemaphoreType.DMA,
        ],
        compiler_params=cp,
    )
    return run(ctx, neg, tgt, wt_in, wt_out).reshape(())


def kernel(context_idxs, target_idx, negative_samples, W_in, W_out):
    return _cbow_loss(context_idxs.astype(jnp.int32),
                      negative_samples.astype(jnp.int32),
                      target_idx.reshape(1).astype(jnp.int32),
                      W_in.T, W_out.T)


# speedup vs baseline: 1.0508x; 1.0508x over previous
import dataclasses

import jax
import jax.numpy as jnp
from jax import lax
from jax.experimental import pallas as pl
from jax.experimental.pallas import tpu as pltpu
from jax.experimental.pallas import tpu_sc as plsc

VOCAB = 1000000
EMBED = 64
N_CTX = 50
N_NEG = 20
LANES = 16
N_SUB = 16
CTX_SLOTS = 4
OUT_SLOTS = 2
SLOTS = CTX_SLOTS + OUT_SLOTS
N_PAD = N_SUB * SLOTS
NEG_BASE = N_SUB * CTX_SLOTS
TGT_ROW = NEG_BASE + 24
PART_BASE = N_PAD
PART_OFF = PART_BASE - NEG_BASE
TGT_OFF = TGT_ROW - NEG_BASE
TILE_L = 128
INV_CTX = 1.0 / N_CTX


def _softplus16(t):
    m = jnp.maximum(t, 0.0)
    u = jnp.exp(-jnp.abs(t))
    y = 1.0 + u
    x = u * (6.0 + u) / (6.0 + 4.0 * u)
    for _ in range(3):
        x = x + y * jnp.exp(-x) - 1.0
    return m + x


def _sc_body(ctx_hbm, neg_hbm, tgt_hbm, wt_in_hbm, wt_out_hbm, out_hbm,
             idx_v, blks_v, col_v, final_v, acc_v, scores_v, out_v,
             stage_shr, sem_a, sem_b):
    cid = lax.axis_index("c")
    sub = lax.axis_index("s")

    @pl.when(cid == 0)
    def _():
        io = lax.iota(jnp.int32, LANES)
        zero16f = jnp.zeros((LANES,), jnp.float32)

        idx_v[pl.ds(N_CTX - 2, LANES)] = jnp.zeros((LANES,), jnp.int32)
        idx_v[pl.ds(NEG_BASE + LANES, LANES)] = jnp.zeros((LANES,), jnp.int32)
        cp_ctx = pltpu.async_copy(ctx_hbm, idx_v.at[pl.ds(0, N_CTX)], sem_b)
        cp_neg = pltpu.async_copy(neg_hbm, idx_v.at[pl.ds(NEG_BASE, N_NEG)],
                                  sem_b)
        cp_tgt = pltpu.async_copy(tgt_hbm, idx_v.at[pl.ds(TGT_ROW, 1)], sem_b)

        def _fire(s, src):
            chunk = idx_v[pl.ds(pl.multiple_of(LANES * s, LANES), LANES)]
            r = jnp.sum(jnp.where(io == sub, chunk, 0))
            q128 = pl.multiple_of((r // TILE_L) * TILE_L, TILE_L)
            pltpu.async_copy(src.at[:, pl.ds(q128, TILE_L)],
                             blks_v.at[s], sem_a)

        live_pred = {3: sub < 2, 5: jnp.logical_or(sub < 4, sub == 8)}

        cp_ctx.wait()

        @pl.loop(0, CTX_SLOTS - 1)
        def _(s):
            _fire(s, wt_in_hbm)

        @pl.when(live_pred[3])
        def _():
            _fire(3, wt_in_hbm)

        cp_neg.wait()
        cp_tgt.wait()
        _fire(4, wt_out_hbm)

        @pl.when(live_pred[5])
        def _():
            _fire(5, wt_out_hbm)

        def _drain(s):
            pltpu.make_async_copy(wt_in_hbm.at[:, pl.ds(0, TILE_L)],
                                  blks_v.at[s], sem_a).wait()

        @pl.loop(0, CTX_SLOTS - 1)
        def _(s):
            _drain(s)

        @pl.when(live_pred[3])
        def _():
            _drain(3)
        _drain(4)

        @pl.when(live_pred[5])
        def _():
            _drain(5)

        def _cols(s):
            chunk = idx_v[pl.ds(pl.multiple_of(LANES * s, LANES), LANES)]
            r = jnp.sum(jnp.where(io == sub, chunk, 0))
            q = r - (r // TILE_L) * TILE_L
            colidx = jnp.full((LANES,), q, jnp.int32)
            return [plsc.load_gather(blks_v.at[s], [io + LANES * c, colidx])
                    for c in range(EMBED // LANES)]

        def _ctx_accum(s):
            vals = _cols(s)
            for c in range(EMBED // LANES):
                sl = pl.ds(LANES * c, LANES)
                acc_v[sl] = acc_v[sl] + vals[c]

        def _wout_stage(s):
            vals = _cols(s)
            for c in range(EMBED // LANES):
                col_v[s, pl.ds(LANES * c, LANES)] = vals[c]
            pltpu.async_copy(col_v.at[s], stage_shr.at[sub + LANES * s],
                             sem_b)

        def _wout_drain(s):
            pltpu.make_async_copy(col_v.at[s],
                                  stage_shr.at[sub + LANES * s],
                                  sem_b).wait()

        for c in range(EMBED // LANES):
            acc_v[pl.ds(LANES * c, LANES)] = zero16f

        @pl.loop(0, CTX_SLOTS - 1)
        def _(s):
            _ctx_accum(s)

        @pl.when(live_pred[3])
        def _():
            _ctx_accum(3)

        pltpu.async_copy(acc_v, stage_shr.at[PART_BASE + sub], sem_b)
        _wout_stage(4)

        @pl.when(live_pred[5])
        def _():
            _wout_stage(5)

        pltpu.make_async_copy(acc_v, stage_shr.at[PART_BASE + sub],
                              sem_b).wait()
        _wout_drain(4)

        @pl.when(live_pred[5])
        def _():
            _wout_drain(5)

        plsc.subcore_barrier()

        @pl.when(sub == 0)
        def _():
            pltpu.sync_copy(stage_shr.at[pl.ds(NEG_BASE, 48)], final_v)
            scores_v[pl.ds(0, LANES)] = zero16f
            scores_v[pl.ds(LANES, LANES)] = zero16f

            hs = [final_v[PART_OFF, pl.ds(LANES * c, LANES)]
                  for c in range(EMBED // LANES)]
            for i in range(1, N_SUB):
                for c in range(EMBED // LANES):
                    hs[c] = hs[c] + final_v[PART_OFF + i,
                                            pl.ds(LANES * c, LANES)]
            for c in range(EMBED // LANES):
                acc_v[pl.ds(LANES * c, LANES)] = hs[c]

            @pl.loop(0, N_NEG)
            def _(j):
                p = final_v[j, pl.ds(0, LANES)] * acc_v[pl.ds(0, LANES)]
                for c in range(1, EMBED // LANES):
                    sl = pl.ds(LANES * c, LANES)
                    p = p + final_v[j, sl] * acc_v[sl]
                s = jnp.sum(p) * INV_CTX
                plsc.store_scatter(scores_v, [jnp.full((LANES,), j, jnp.int32)],
                                   jnp.full((LANES,), s, jnp.float32),
                                   mask=io == 0)

            p = final_v[TGT_OFF, pl.ds(0, LANES)] * acc_v[pl.ds(0, LANES)]
            for c in range(1, EMBED // LANES):
                sl = pl.ds(LANES * c, LANES)
                p = p + final_v[TGT_OFF, sl] * acc_v[sl]
            s_pos = jnp.sum(p) * INV_CTX

            neg100 = jnp.full((LANES,), -100.0, jnp.float32)
            t0 = scores_v[pl.ds(0, LANES)]
            t1 = jnp.where(io < (N_NEG - LANES), scores_v[pl.ds(LANES, LANES)],
                           jnp.where(io == (N_NEG - LANES), -s_pos, neg100))
            loss = jnp.sum(_softplus16(t0)) + jnp.sum(_softplus16(t1))
            out_v[...] = jnp.full((LANES,), loss, jnp.float32)
            pltpu.sync_copy(out_v.at[pl.ds(0, 1)], out_hbm)


@jax.jit
def _cbow_loss(ctx, neg, tgt, wt_in, wt_out):
    mesh = plsc.VectorSubcoreMesh(core_axis_name="c", subcore_axis_name="s",
                                  num_cores=1, num_subcores=N_SUB)
    cp = pltpu.CompilerParams()
    if "needs_layout_passes" in pltpu.CompilerParams.__dataclass_fields__:
        cp = dataclasses.replace(cp, needs_layout_passes=False)
    cp = dataclasses.replace(cp, disable_bounds_checks=True)
    run = pl.kernel(
        _sc_body,
        out_type=jax.ShapeDtypeStruct((1,), jnp.float32),
        mesh=mesh,
        scratch_types=[
            pltpu.VMEM((N_PAD,), jnp.int32),
            pltpu.VMEM((SLOTS, EMBED, TILE_L), jnp.float32),
            pltpu.VMEM((SLOTS, EMBED), jnp.float32),
            pltpu.VMEM((48, EMBED), jnp.float32),
            pltpu.VMEM((EMBED,), jnp.float32),
            pltpu.VMEM((2 * LANES,), jnp.float32),
            pltpu.VMEM((LANES,), jnp.float32),
            pltpu.VMEM_SHARED((PART_BASE + N_SUB, EMBED), jnp.float32),
            pltpu.SemaphoreType.DMA,
            pltpu.SemaphoreType.DMA,
        ],
        compiler_params=cp,
    )
    return run(ctx, neg, tgt, wt_in, wt_out).reshape(())


def kernel(context_idxs, target_idx, negative_samples, W_in, W_out):
    return _cbow_loss(context_idxs.astype(jnp.int32),
                      negative_samples.astype(jnp.int32),
                      target_idx.reshape(1).astype(jnp.int32),
                      W_in.T, W_out.T)


# speedup vs baseline: 1.0519x; 1.0011x over previous
import dataclasses

import jax
import jax.numpy as jnp
from jax import lax
from jax.experimental import pallas as pl
from jax.experimental.pallas import tpu as pltpu
from jax.experimental.pallas import tpu_sc as plsc

VOCAB = 1000000
EMBED = 64
N_CTX = 50
N_NEG = 20
LANES = 16
N_SUB = 16
CTX_SLOTS = 4
OUT_SLOTS = 2
SLOTS = CTX_SLOTS + OUT_SLOTS
N_PAD = N_SUB * SLOTS
NEG_BASE = N_SUB * CTX_SLOTS
TGT_ROW = NEG_BASE + 24
PART_BASE = N_PAD
PART_OFF = PART_BASE - NEG_BASE
TGT_OFF = TGT_ROW - NEG_BASE
TILE_L = 128
INV_CTX = 1.0 / N_CTX


def _softplus16(t):
    m = jnp.maximum(t, 0.0)
    u = jnp.exp(-jnp.abs(t))
    y = 1.0 + u
    x = u * (6.0 + u) / (6.0 + 4.0 * u)
    for _ in range(3):
        x = x + y * jnp.exp(-x) - 1.0
    return m + x


def _sc_body(ctx_hbm, neg_hbm, tgt_hbm, wt_in_hbm, wt_out_hbm, out_hbm,
             idx_v, blks_v, col_v, final_v, acc_v, scores_v, out_v,
             stage_shr, sem_a, sem_b):
    cid = lax.axis_index("c")
    sub = lax.axis_index("s")

    @pl.when(cid == 0)
    def _():
        io = lax.iota(jnp.int32, LANES)
        zero16f = jnp.zeros((LANES,), jnp.float32)

        idx_v[pl.ds(N_CTX - 2, LANES)] = jnp.zeros((LANES,), jnp.int32)
        idx_v[pl.ds(NEG_BASE + LANES, LANES)] = jnp.zeros((LANES,), jnp.int32)
        cp_ctx = pltpu.async_copy(ctx_hbm, idx_v.at[pl.ds(0, N_CTX)], sem_b)
        cp_neg = pltpu.async_copy(neg_hbm, idx_v.at[pl.ds(NEG_BASE, N_NEG)],
                                  sem_b)
        cp_tgt = pltpu.async_copy(tgt_hbm, idx_v.at[pl.ds(TGT_ROW, 1)], sem_b)

        def _fire(s, src):
            chunk = idx_v[pl.ds(pl.multiple_of(LANES * s, LANES), LANES)]
            r = jnp.sum(jnp.where(io == sub, chunk, 0))
            q128 = pl.multiple_of((r // TILE_L) * TILE_L, TILE_L)
            pltpu.async_copy(src.at[:, pl.ds(q128, TILE_L)],
                             blks_v.at[s], sem_a)

        live_pred = {3: sub < 2, 5: jnp.logical_or(sub < 4, sub == 8)}

        cp_ctx.wait()

        @pl.loop(0, CTX_SLOTS - 1)
        def _(s):
            _fire(s, wt_in_hbm)

        @pl.when(live_pred[3])
        def _():
            _fire(3, wt_in_hbm)

        cp_neg.wait()
        cp_tgt.wait()
        _fire(4, wt_out_hbm)

        @pl.when(live_pred[5])
        def _():
            _fire(5, wt_out_hbm)

        def _drain(s):
            pltpu.make_async_copy(wt_in_hbm.at[:, pl.ds(0, TILE_L)],
                                  blks_v.at[s], sem_a).wait()

        @pl.loop(0, CTX_SLOTS - 1)
        def _(s):
            _drain(s)

        @pl.when(live_pred[3])
        def _():
            _drain(3)
        _drain(4)

        @pl.when(live_pred[5])
        def _():
            _drain(5)

        def _cols(s):
            chunk = idx_v[pl.ds(pl.multiple_of(LANES * s, LANES), LANES)]
            r = jnp.sum(jnp.where(io == sub, chunk, 0))
            q = r - (r // TILE_L) * TILE_L
            colidx = jnp.full((LANES,), q, jnp.int32)
            return [plsc.load_gather(blks_v.at[s], [io + LANES * c, colidx])
                    for c in range(EMBED // LANES)]

        def _ctx_accum(s):
            vals = _cols(s)
            for c in range(EMBED // LANES):
                sl = pl.ds(LANES * c, LANES)
                acc_v[sl] = acc_v[sl] + vals[c]

        def _wout_stage(s):
            vals = _cols(s)
            for c in range(EMBED // LANES):
                col_v[s, pl.ds(LANES * c, LANES)] = vals[c]
            pltpu.async_copy(col_v.at[s], stage_shr.at[sub + LANES * s],
                             sem_b)

        def _wout_drain(s):
            pltpu.make_async_copy(col_v.at[s],
                                  stage_shr.at[sub + LANES * s],
                                  sem_b).wait()

        for c in range(EMBED // LANES):
            acc_v[pl.ds(LANES * c, LANES)] = zero16f

        @pl.loop(0, CTX_SLOTS - 1)
        def _(s):
            _ctx_accum(s)

        @pl.when(live_pred[3])
        def _():
            _ctx_accum(3)

        pltpu.async_copy(acc_v, stage_shr.at[PART_BASE + sub], sem_b)
        _wout_stage(4)

        @pl.when(live_pred[5])
        def _():
            _wout_stage(5)

        pltpu.make_async_copy(acc_v, stage_shr.at[PART_BASE + sub],
                              sem_b).wait()
        _wout_drain(4)

        @pl.when(live_pred[5])
        def _():
            _wout_drain(5)

        plsc.subcore_barrier()

        @pl.when(sub == 0)
        def _():
            pltpu.sync_copy(stage_shr.at[pl.ds(NEG_BASE, 48)], final_v)
            scores_v[pl.ds(0, LANES)] = zero16f
            scores_v[pl.ds(LANES, LANES)] = zero16f

            hs = [final_v[PART_OFF, pl.ds(LANES * c, LANES)]
                  for c in range(EMBED // LANES)]
            for i in range(1, N_SUB):
                for c in range(EMBED // LANES):
                    hs[c] = hs[c] + final_v[PART_OFF + i,
                                            pl.ds(LANES * c, LANES)]
            for c in range(EMBED // LANES):
                acc_v[pl.ds(LANES * c, LANES)] = hs[c]

            @pl.loop(0, N_NEG)
            def _(j):
                p = final_v[j, pl.ds(0, LANES)] * acc_v[pl.ds(0, LANES)]
                for c in range(1, EMBED // LANES):
                    sl = pl.ds(LANES * c, LANES)
                    p = p + final_v[j, sl] * acc_v[sl]
                s = jnp.sum(p) * INV_CTX
                plsc.store_scatter(scores_v, [jnp.full((LANES,), j, jnp.int32)],
                                   jnp.full((LANES,), s, jnp.float32),
                                   mask=io == 0)

            p = final_v[TGT_OFF, pl.ds(0, LANES)] * acc_v[pl.ds(0, LANES)]
            for c in range(1, EMBED // LANES):
                sl = pl.ds(LANES * c, LANES)
                p = p + final_v[TGT_OFF, sl] * acc_v[sl]
            s_pos = jnp.sum(p) * INV_CTX

            neg100 = jnp.full((LANES,), -100.0, jnp.float32)
            t0 = scores_v[pl.ds(0, LANES)]
            t1 = jnp.where(io < (N_NEG - LANES), scores_v[pl.ds(LANES, LANES)],
                           jnp.where(io == (N_NEG - LANES), -s_pos, neg100))
            loss = jnp.sum(_softplus16(t0)) + jnp.sum(_softplus16(t1))
            out_v[...] = jnp.full((LANES,), loss, jnp.float32)
            pltpu.sync_copy(out_v.at[pl.ds(0, 1)], out_hbm)


@jax.jit
def _cbow_loss(ctx, neg, tgt, wt_in, wt_out):
    mesh = plsc.VectorSubcoreMesh(core_axis_name="c", subcore_axis_name="s",
                                  num_cores=1, num_subcores=N_SUB)
    cp = pltpu.CompilerParams()
    if "needs_layout_passes" in pltpu.CompilerParams.__dataclass_fields__:
        cp = dataclasses.replace(cp, needs_layout_passes=False)
    cp = dataclasses.replace(cp, disable_bounds_checks=True,
                             skip_device_barrier=True)
    run = pl.kernel(
        _sc_body,
        out_type=jax.ShapeDtypeStruct((1,), jnp.float32),
        mesh=mesh,
        scratch_types=[
            pltpu.VMEM((N_PAD,), jnp.int32),
            pltpu.VMEM((SLOTS, EMBED, TILE_L), jnp.float32),
            pltpu.VMEM((SLOTS, EMBED), jnp.float32),
            pltpu.VMEM((48, EMBED), jnp.float32),
            pltpu.VMEM((EMBED,), jnp.float32),
            pltpu.VMEM((2 * LANES,), jnp.float32),
            pltpu.VMEM((LANES,), jnp.float32),
            pltpu.VMEM_SHARED((PART_BASE + N_SUB, EMBED), jnp.float32),
            pltpu.SemaphoreType.DMA,
            pltpu.SemaphoreType.DMA,
        ],
        compiler_params=cp,
    )
    return run(ctx, neg, tgt, wt_in, wt_out).reshape(())


def kernel(context_idxs, target_idx, negative_samples, W_in, W_out):
    return _cbow_loss(context_idxs.astype(jnp.int32),
                      negative_samples.astype(jnp.int32),
                      target_idx.reshape(1).astype(jnp.int32),
                      W_in.T, W_out.T)


# speedup vs baseline: 1.0538x; 1.0018x over previous
import dataclasses

import jax
import jax.numpy as jnp
from jax import lax
from jax.experimental import pallas as pl
from jax.experimental.pallas import tpu as pltpu
from jax.experimental.pallas import tpu_sc as plsc

VOCAB = 1000000
EMBED = 64
N_CTX = 50
N_NEG = 20
LANES = 16
N_SUB = 16
CTX_SLOTS = 4
OUT_SLOTS = 2
SLOTS = CTX_SLOTS + OUT_SLOTS
N_PAD = N_SUB * SLOTS
NEG_BASE = N_SUB * CTX_SLOTS
TGT_ROW = NEG_BASE + 24
PART_BASE = N_PAD
PART_OFF = PART_BASE - NEG_BASE
TGT_OFF = TGT_ROW - NEG_BASE
TILE_L = 128
INV_CTX = 1.0 / N_CTX


def _softplus16(t):
    m = jnp.maximum(t, 0.0)
    u = jnp.exp(-jnp.abs(t))
    y = 1.0 + u
    x = u * (6.0 + u) / (6.0 + 4.0 * u)
    for _ in range(3):
        x = x + y * jnp.exp(-x) - 1.0
    return m + x


def _sc_body(ctx_hbm, neg_hbm, tgt_hbm, wt_in_hbm, wt_out_hbm, out_hbm,
             idx_v, blks_v, col_v, final_v, acc_v, scores_v, out_v,
             stage_shr, sem_a, sem_b):
    cid = lax.axis_index("c")
    sub = lax.axis_index("s")

    @pl.when(cid == 0)
    def _():
        io = lax.iota(jnp.int32, LANES)
        zero16f = jnp.zeros((LANES,), jnp.float32)

        idx_v[pl.ds(N_CTX - 2, LANES)] = jnp.zeros((LANES,), jnp.int32)
        idx_v[pl.ds(NEG_BASE + LANES, LANES)] = jnp.zeros((LANES,), jnp.int32)
        cp_ctx = pltpu.async_copy(ctx_hbm, idx_v.at[pl.ds(0, N_CTX)], sem_b)
        cp_neg = pltpu.async_copy(neg_hbm, idx_v.at[pl.ds(NEG_BASE, N_NEG)],
                                  sem_b)
        cp_tgt = pltpu.async_copy(tgt_hbm, idx_v.at[pl.ds(TGT_ROW, 1)], sem_b)

        def _fire(s, src):
            chunk = idx_v[pl.ds(pl.multiple_of(LANES * s, LANES), LANES)]
            r = jnp.sum(jnp.where(io == sub, chunk, 0))
            q128 = pl.multiple_of((r // TILE_L) * TILE_L, TILE_L)
            pltpu.async_copy(src.at[:, pl.ds(q128, TILE_L)],
                             blks_v.at[s], sem_a)

        live_pred = {3: sub < 2, 5: jnp.logical_or(sub < 4, sub == 8)}

        cp_ctx.wait()

        @pl.loop(0, CTX_SLOTS - 1)
        def _(s):
            _fire(s, wt_in_hbm)

        @pl.when(live_pred[3])
        def _():
            _fire(3, wt_in_hbm)

        cp_neg.wait()
        cp_tgt.wait()
        _fire(4, wt_out_hbm)

        @pl.when(live_pred[5])
        def _():
            _fire(5, wt_out_hbm)

        def _drain(s):
            pltpu.make_async_copy(wt_in_hbm.at[:, pl.ds(0, TILE_L)],
                                  blks_v.at[s], sem_a).wait()

        @pl.loop(0, CTX_SLOTS - 1)
        def _(s):
            _drain(s)

        @pl.when(live_pred[3])
        def _():
            _drain(3)
        _drain(4)

        @pl.when(live_pred[5])
        def _():
            _drain(5)

        def _cols(s):
            chunk = idx_v[pl.ds(pl.multiple_of(LANES * s, LANES), LANES)]
            r = jnp.sum(jnp.where(io == sub, chunk, 0))
            q = r - (r // TILE_L) * TILE_L
            colidx = jnp.full((LANES,), q, jnp.int32)
            return [plsc.load_gather(blks_v.at[s], [io + LANES * c, colidx])
                    for c in range(EMBED // LANES)]

        def _ctx_accum(s):
            vals = _cols(s)
            for c in range(EMBED // LANES):
                sl = pl.ds(LANES * c, LANES)
                acc_v[sl] = acc_v[sl] + vals[c]

        def _wout_stage(s):
            vals = _cols(s)
            for c in range(EMBED // LANES):
                col_v[s, pl.ds(LANES * c, LANES)] = vals[c]
            pltpu.async_copy(col_v.at[s], stage_shr.at[sub + LANES * s],
                             sem_b)

        def _wout_drain(s):
            pltpu.make_async_copy(col_v.at[s],
                                  stage_shr.at[sub + LANES * s],
                                  sem_b).wait()

        for c in range(EMBED // LANES):
            acc_v[pl.ds(LANES * c, LANES)] = zero16f

        @pl.loop(0, CTX_SLOTS - 1)
        def _(s):
            _ctx_accum(s)

        @pl.when(live_pred[3])
        def _():
            _ctx_accum(3)

        pltpu.async_copy(acc_v, stage_shr.at[PART_BASE + sub], sem_b)
        _wout_stage(4)

        @pl.when(live_pred[5])
        def _():
            _wout_stage(5)

        pltpu.make_async_copy(acc_v, stage_shr.at[PART_BASE + sub],
                              sem_b).wait()
        _wout_drain(4)

        @pl.when(live_pred[5])
        def _():
            _wout_drain(5)

        plsc.subcore_barrier()

        @pl.when(sub == 0)
        def _():
            pltpu.sync_copy(stage_shr.at[pl.ds(NEG_BASE, 48)], final_v)
            scores_v[pl.ds(0, LANES)] = zero16f
            scores_v[pl.ds(LANES, LANES)] = zero16f

            hs = [final_v[PART_OFF, pl.ds(LANES * c, LANES)]
                  for c in range(EMBED // LANES)]
            for i in range(1, N_SUB):
                for c in range(EMBED // LANES):
                    hs[c] = hs[c] + final_v[PART_OFF + i,
                                            pl.ds(LANES * c, LANES)]
            for c in range(EMBED // LANES):
                acc_v[pl.ds(LANES * c, LANES)] = hs[c]

            @pl.loop(0, N_NEG)
            def _(j):
                p = final_v[j, pl.ds(0, LANES)] * acc_v[pl.ds(0, LANES)]
                for c in range(1, EMBED // LANES):
                    sl = pl.ds(LANES * c, LANES)
                    p = p + final_v[j, sl] * acc_v[sl]
                s = jnp.sum(p) * INV_CTX
                plsc.store_scatter(scores_v, [jnp.full((LANES,), j, jnp.int32)],
                                   jnp.full((LANES,), s, jnp.float32),
                                   mask=io == 0)

            p = final_v[TGT_OFF, pl.ds(0, LANES)] * acc_v[pl.ds(0, LANES)]
            for c in range(1, EMBED // LANES):
                sl = pl.ds(LANES * c, LANES)
                p = p + final_v[TGT_OFF, sl] * acc_v[sl]
            s_pos = jnp.sum(p) * INV_CTX

            neg100 = jnp.full((LANES,), -100.0, jnp.float32)
            t0 = scores_v[pl.ds(0, LANES)]
            t1 = jnp.where(io < (N_NEG - LANES), scores_v[pl.ds(LANES, LANES)],
                           jnp.where(io == (N_NEG - LANES), -s_pos, neg100))
            loss = jnp.sum(_softplus16(t0)) + jnp.sum(_softplus16(t1))
            out_v[...] = jnp.full((LANES,), loss, jnp.float32)
            pltpu.sync_copy(out_v.at[pl.ds(0, 1)], out_hbm)


@jax.jit
def _cbow_loss(ctx, neg, tgt, wt_in, wt_out):
    mesh = plsc.VectorSubcoreMesh(core_axis_name="c", subcore_axis_name="s",
                                  num_cores=1, num_subcores=N_SUB)
    cp = pltpu.CompilerParams()
    if "needs_layout_passes" in pltpu.CompilerParams.__dataclass_fields__:
        cp = dataclasses.replace(cp, needs_layout_passes=False)
    cp = dataclasses.replace(cp, disable_bounds_checks=True)
    run = pl.kernel(
        _sc_body,
        out_type=jax.ShapeDtypeStruct((1,), jnp.float32),
        mesh=mesh,
        scratch_types=[
            pltpu.VMEM((N_PAD,), jnp.int32),
            pltpu.VMEM((SLOTS, EMBED, TILE_L), jnp.float32),
            pltpu.VMEM((SLOTS, EMBED), jnp.float32),
            pltpu.VMEM((48, EMBED), jnp.float32),
            pltpu.VMEM((EMBED,), jnp.float32),
            pltpu.VMEM((2 * LANES,), jnp.float32),
            pltpu.VMEM((LANES,), jnp.float32),
            pltpu.VMEM_SHARED((PART_BASE + N_SUB, EMBED), jnp.float32),
            pltpu.SemaphoreType.DMA,
            pltpu.SemaphoreType.DMA,
        ],
        compiler_params=cp,
    )
    return run(ctx, neg, tgt, wt_in, wt_out).reshape(())


def kernel(context_idxs, target_idx, negative_samples, W_in, W_out):
    return _cbow_loss(context_idxs.astype(jnp.int32),
                      negative_samples.astype(jnp.int32),
                      target_idx.reshape(1).astype(jnp.int32),
                      W_in.T, W_out.T)
